# baseline (speedup 1.0000x reference)
import jax
import jax.numpy as jnp
from jax.experimental import pallas as pl
from jax.experimental.pallas import tpu as pltpu


def _selected_rows(C, bank_size):
    key = jax.random.key(42)
    fg_rows = jax.random.permutation(key, bank_size - 1)[: C - 1] + 1
    bg_rows = jnp.zeros((1,), dtype=fg_rows.dtype)
    return jnp.concatenate([bg_rows, fg_rows])


def _encode_body(rows_ref, pos_ref, dense_ref, sparse_ref,
                 dense_out_ref, sparse_out_ref):
    c = pl.program_id(2)
    row = rows_ref[c]
    enc = pos_ref[0, 0, row, :]
    dense_out_ref[...] = dense_ref[...] + enc[None, None, None, :, None, None]
    sparse_out_ref[...] = sparse_ref[...] + enc[None, None, None, None, :]


def kernel(dense_embeddings, sparse_embeddings, pos_embedding):
    B, M, C, N, D = sparse_embeddings.shape
    _, _, _, _, H, W = dense_embeddings.shape
    bank_size = pos_embedding.shape[2]

    rows = _selected_rows(C, bank_size).astype(jnp.int32)

    grid_spec = pltpu.PrefetchScalarGridSpec(
        num_scalar_prefetch=1,
        grid=(B, M, C),
        in_specs=[
            pl.BlockSpec((1, 1, bank_size, D), lambda b, m, c, rr: (0, 0, 0, 0)),
            pl.BlockSpec((1, 1, 1, D, H, W), lambda b, m, c, rr: (b, m, c, 0, 0, 0)),
            pl.BlockSpec((1, 1, 1, N, D), lambda b, m, c, rr: (b, m, c, 0, 0)),
        ],
        out_specs=[
            pl.BlockSpec((1, 1, 1, D, H, W), lambda b, m, c, rr: (b, m, c, 0, 0, 0)),
            pl.BlockSpec((1, 1, 1, N, D), lambda b, m, c, rr: (b, m, c, 0, 0)),
        ],
    )

    dense_out, sparse_out = pl.pallas_call(
        _encode_body,
        grid_spec=grid_spec,
        out_shape=[
            jax.ShapeDtypeStruct((B, M, C, D, H, W), jnp.float32),
            jax.ShapeDtypeStruct((B, M, C, N, D), jnp.float32),
        ],
        compiler_params=pltpu.CompilerParams(
            dimension_semantics=("parallel", "parallel", "arbitrary"),
        ),
    )(rows, pos_embedding, dense_embeddings, sparse_embeddings)

    return (dense_out, sparse_out)


# speedup vs baseline: 6.8751x; 6.8751x over previous
import jax
import jax.numpy as jnp
from jax.experimental import pallas as pl
from jax.experimental.pallas import tpu as pltpu


def _selected_rows(C, bank_size):
    key = jax.random.key(42)
    fg_rows = jax.random.permutation(key, bank_size - 1)[: C - 1] + 1
    bg_rows = jnp.zeros((1,), dtype=fg_rows.dtype)
    return jnp.concatenate([bg_rows, fg_rows])


def _encode_body(rowmap_ref, pos_ref, dense_ref, sparse_ref,
                 dense_out_ref, sparse_out_ref):
    i = pl.program_id(0)
    row = rowmap_ref[i]
    enc = pos_ref[row, :]
    dense_out_ref[...] = dense_ref[...] + enc[None, None, :]
    sparse_out_ref[...] = sparse_ref[...] + enc[None, None, :]


def kernel(dense_embeddings, sparse_embeddings, pos_embedding):
    B, M, C, N, D = sparse_embeddings.shape
    _, _, _, _, H, W = dense_embeddings.shape
    bank_size = pos_embedding.shape[2]
    G = B * M * C
    HW = H * W

    rows = _selected_rows(C, bank_size).astype(jnp.int32)
    rowmap = jnp.tile(rows, B * M)

    dense3 = dense_embeddings.transpose(0, 1, 2, 4, 5, 3).reshape(G, HW, D)
    sparse3 = sparse_embeddings.reshape(G, N, D)
    pos2 = pos_embedding.reshape(bank_size, D)

    grid_spec = pltpu.PrefetchScalarGridSpec(
        num_scalar_prefetch=1,
        grid=(G,),
        in_specs=[
            pl.BlockSpec((bank_size, D), lambda i, rm: (0, 0)),
            pl.BlockSpec((1, HW, D), lambda i, rm: (i, 0, 0)),
            pl.BlockSpec((1, N, D), lambda i, rm: (i, 0, 0)),
        ],
        out_specs=[
            pl.BlockSpec((1, HW, D), lambda i, rm: (i, 0, 0)),
            pl.BlockSpec((1, N, D), lambda i, rm: (i, 0, 0)),
        ],
    )

    dense_out, sparse_out = pl.pallas_call(
        _encode_body,
        grid_spec=grid_spec,
        out_shape=[
            jax.ShapeDtypeStruct((G, HW, D), jnp.float32),
            jax.ShapeDtypeStruct((G, N, D), jnp.float32),
        ],
        compiler_params=pltpu.CompilerParams(
            dimension_semantics=("arbitrary",),
        ),
    )(rowmap, pos2, dense3, sparse3)

    dense_out = dense_out.reshape(B, M, C, H, W, D).transpose(0, 1, 2, 5, 3, 4)
    return (dense_out, sparse_out.reshape(B, M, C, N, D))


# speedup vs baseline: 7.1529x; 1.0404x over previous
import jax
import jax.numpy as jnp
import numpy as np
from jax.experimental import pallas as pl
from jax.experimental.pallas import tpu as pltpu


def _selected_rows(C, bank_size):
    key = jax.random.key(42)
    fg_rows = jax.random.permutation(key, bank_size - 1)[: C - 1] + 1
    bg_rows = jnp.zeros((1,), dtype=fg_rows.dtype)
    return jnp.concatenate([bg_rows, fg_rows])


def _encode_body(rowmap_ref, pos_ref, dense_ref, sparse_ref,
                 dense_out_ref, sparse_out_ref):
    i = pl.program_id(0)
    row = rowmap_ref[i]
    enc = pos_ref[row, :]
    dense_out_ref[...] = dense_ref[...] + enc[None, None, :]
    sparse_out_ref[...] = sparse_ref[...] + enc[None, None, :]


def kernel(dense_embeddings, sparse_embeddings, pos_embedding):
    B, M, C, N, D = sparse_embeddings.shape
    _, _, _, _, H, W = dense_embeddings.shape
    bank_size = pos_embedding.shape[2]
    G = B * M * C
    HW = H * W

    try:
        with jax.ensure_compile_time_eval():
            rows_np = np.asarray(_selected_rows(C, bank_size)).astype(np.int32)
        rowmap = jnp.asarray(np.tile(rows_np, B * M))
    except Exception:
        rows = _selected_rows(C, bank_size).astype(jnp.int32)
        rowmap = jnp.tile(rows, B * M)

    dense3 = dense_embeddings.transpose(0, 1, 2, 4, 5, 3).reshape(G, HW, D)
    sparse3 = sparse_embeddings.reshape(G, N, D)
    pos2 = pos_embedding.reshape(bank_size, D)

    grid_spec = pltpu.PrefetchScalarGridSpec(
        num_scalar_prefetch=1,
        grid=(G,),
        in_specs=[
            pl.BlockSpec((bank_size, D), lambda i, rm: (0, 0)),
            pl.BlockSpec((1, HW, D), lambda i, rm: (i, 0, 0)),
            pl.BlockSpec((1, N, D), lambda i, rm: (i, 0, 0)),
        ],
        out_specs=[
            pl.BlockSpec((1, HW, D), lambda i, rm: (i, 0, 0)),
            pl.BlockSpec((1, N, D), lambda i, rm: (i, 0, 0)),
        ],
    )

    dense_out, sparse_out = pl.pallas_call(
        _encode_body,
        grid_spec=grid_spec,
        out_shape=[
            jax.ShapeDtypeStruct((G, HW, D), jnp.float32),
            jax.ShapeDtypeStruct((G, N, D), jnp.float32),
        ],
        compiler_params=pltpu.CompilerParams(
            dimension_semantics=("arbitrary",),
        ),
    )(rowmap, pos2, dense3, sparse3)

    dense_out = dense_out.reshape(B, M, C, H, W, D).transpose(0, 1, 2, 5, 3, 4)
    return (dense_out, sparse_out.reshape(B, M, C, N, D))
